# baseline (speedup 1.0000x reference)
import jax
import jax.numpy as jnp
from jax import lax
from jax.experimental import pallas as pl
from jax.experimental.pallas import tpu as pltpu
from jax.experimental.pallas import tpu_sc as plsc

_NC = 2
_NS = 16
_NW = _NC * _NS
_ICH = 128
_L = 16

_EPS = 1e-5


def _sc_gather_body(ids_hbm, name_hbm, eid_hbm, ename_hbm,
                    out_id, out_name,
                    idx_v, pair_v, pairs, rows, sem0, sem1):
    bpw = idx_v.shape[0]
    nch = bpw // _ICH
    ngrp = _ICH // _L
    wid = lax.axis_index("s") * _NC + lax.axis_index("c")
    base = wid * bpw
    lane = jax.lax.iota(jnp.int32, _L)

    for idx_hbm, tbl_hbm, out_hbm in ((ids_hbm, eid_hbm, out_id),
                                      (name_hbm, ename_hbm, out_name)):
        pltpu.sync_copy(idx_hbm.at[wid], idx_v)

        def mk_pairs(g, carry):
            ig = idx_v[pl.ds(g * _L, _L)]
            pair_v[pl.ds(g * _L, _L)] = lax.shift_right_logical(ig, 1)
            return carry

        lax.fori_loop(0, bpw // _L, mk_pairs, 0, unroll=False)

        def fire(c):
            return pltpu.make_async_copy(
                tbl_hbm.at[pair_v.at[pl.ds(c * _ICH, _ICH)]],
                pairs.at[c % 2], sem0 if c % 2 == 0 else sem1)

        def extract(c):
            buf = pairs.at[c % 2]

            def grp(g, carry):
                ig = idx_v[pl.ds(c * _ICH + g * _L, _L)]
                off = lax.mul(lax.bitwise_and(ig, 1), 64)
                slot = lax.add(lax.mul(g, _L), lane)
                dst = lax.add(lax.full((_L,), c * _ICH, jnp.int32), slot)
                for j in range(64):
                    jv = lax.full((_L,), j, jnp.int32)
                    v = plsc.load_gather(buf, [slot, lax.add(off, jv)])
                    plsc.store_scatter(rows, [dst, jv], v)
                return carry

            lax.fori_loop(0, ngrp, grp, 0, unroll=False)

        fire(0).start()
        for c in range(nch):
            if c + 1 < nch:
                fire(c + 1).start()
            fire(c).wait()
            extract(c)
        pltpu.sync_copy(rows, out_hbm.at[pl.ds(base, bpw)])


def _sc_gather(ids2, name2, eid_p, ename_p, B, D):
    bpw = B // _NW
    mesh = plsc.VectorSubcoreMesh(core_axis_name="c", subcore_axis_name="s")
    out_sd = jax.ShapeDtypeStruct((B, D), jnp.float32)
    f = pl.kernel(
        _sc_gather_body,
        out_type=(out_sd, out_sd),
        mesh=mesh,
        scratch_types=[
            pltpu.VMEM((bpw,), jnp.int32),
            pltpu.VMEM((bpw,), jnp.int32),
            pltpu.VMEM((2, _ICH, 2 * D), jnp.float32),
            pltpu.VMEM((bpw, D), jnp.float32),
            pltpu.SemaphoreType.DMA,
            pltpu.SemaphoreType.DMA,
        ],
        compiler_params=pltpu.CompilerParams(needs_layout_passes=False),
    )
    return f(ids2, name2, eid_p, ename_p)


def _ln(x):
    mu = jnp.mean(x, axis=-1, keepdims=True)
    var = jnp.mean((x - mu) * (x - mu), axis=-1, keepdims=True)
    return (x - mu) * lax.rsqrt(var + _EPS)


def _gelu(x):
    return x * 0.5 * (1.0 + lax.erf(x * 0.7071067811865476))


def _mlp_body(id_ref, name_ref, cty_ref, ecty_ref, w1a_ref, w1b_ref, w1c_ref,
              b1_ref, w2_ref, b2_ref, w3_ref, b3_ref, out_ref):
    f32 = jnp.float32
    cty = cty_ref[0]
    ncty = ecty_ref.shape[0]
    onehot = jnp.where(
        cty.reshape(-1, 1) == lax.broadcasted_iota(jnp.int32, (1, ncty), 1),
        f32(1.0), f32(0.0))
    cty_emb = jnp.dot(onehot, ecty_ref[...], preferred_element_type=f32)
    h = (jnp.dot(id_ref[...], w1a_ref[...], preferred_element_type=f32)
         + jnp.dot(cty_emb, w1b_ref[...], preferred_element_type=f32)
         + jnp.dot(name_ref[...], w1c_ref[...], preferred_element_type=f32)
         + b1_ref[...])
    h = _gelu(_ln(h))
    h = jnp.dot(h, w2_ref[...], preferred_element_type=f32) + b2_ref[...]
    h = _gelu(_ln(h))
    h = jnp.dot(h, w3_ref[...], preferred_element_type=f32) + b3_ref[...]
    out_ref[...] = _gelu(h)


def _mlp(id_emb, name_emb, cty3, E_cty, W1, b1, W2, b2, W3, b3, block_b):
    B, D = id_emb.shape
    NCTY = E_cty.shape[0]
    H1 = W1.shape[1]
    H2 = W2.shape[1]
    H3 = W3.shape[1]
    grid = (B // block_b,)
    data = lambda: pl.BlockSpec((block_b, D), lambda i: (i, 0))
    full = lambda r, c: pl.BlockSpec((r, c), lambda i: (0, 0))
    return pl.pallas_call(
        _mlp_body,
        grid=grid,
        in_specs=[
            data(), data(),
            pl.BlockSpec((1, 1, block_b), lambda i: (i, 0, 0)),
            full(NCTY, D),
            full(D, H1), full(D, H1), full(D, H1), full(1, H1),
            full(H1, H2), full(1, H2),
            full(H2, H3), full(1, H3),
        ],
        out_specs=pl.BlockSpec((block_b, H3), lambda i: (i, 0)),
        out_shape=jax.ShapeDtypeStruct((B, H3), jnp.float32),
    )(id_emb, name_emb, cty3, E_cty,
      W1[:D], W1[D:2 * D], W1[2 * D:], b1.reshape(1, H1),
      W2, b2.reshape(1, H2), W3, b3.reshape(1, H3))


def kernel(user_ids, user_countries, user_names, E_id, E_cty, E_name,
           W1, b1, W2, b2, W3, b3):
    B = user_ids.shape[0]
    D = E_id.shape[1]
    block_b = 2048
    ids2 = user_ids.astype(jnp.int32).reshape(_NW, -1)
    name2 = user_names.astype(jnp.int32).reshape(_NW, -1)
    cty3 = user_countries.astype(jnp.int32).reshape(B // block_b, 1, block_b)
    eid_p = E_id.reshape(-1, 2 * D)
    ename_p = E_name.reshape(-1, 2 * D)
    id_emb, name_emb = _sc_gather(ids2, name2, eid_p, ename_p, B, D)
    return _mlp(id_emb, name_emb, cty3, E_cty, W1, b1, W2, b2, W3, b3, block_b)
